# baseline (speedup 1.0000x reference)
import functools

import jax
import jax.numpy as jnp
from jax import lax
from jax.experimental import pallas as pl
from jax.experimental.pallas import tpu as pltpu
from jax.experimental.pallas import tpu_sc as plsc

N = 10000
E = 320000
D = 128

NC = 2
NS = 16
NW = NC * NS

NBUF = 4
LEAD = 2
NI = 8
LI = 4
EPT = 10240
EPAD = NW * EPT
CH1, NCH1 = 64, 160
CH2, NCH2 = 80, 128

RPT = 640
NPAD = NS * RPT
CPO = 64
NHOP = RPT // CPO


@functools.lru_cache(maxsize=None)
def _make_segsum(with_cnt: bool, CHUNK: int, NCH: int):
    _mesh = plsc.VectorSubcoreMesh(core_axis_name="c", subcore_axis_name="s",
                                   num_cores=NC, num_subcores=NS)
    out_type = [jax.ShapeDtypeStruct((NC, NPAD, D), jnp.float32)]
    scratch = [
        pltpu.VMEM_SHARED((NPAD, D), jnp.float32),
    ]
    if with_cnt:
        out_type.append(jax.ShapeDtypeStruct((NC, NPAD, 1), jnp.float32))
        scratch.append(pltpu.VMEM_SHARED((NPAD, 16), jnp.float32))

    @functools.partial(pl.kernel, out_type=out_type, mesh=_mesh,
                       scratch_types=scratch,
                       compiler_params=pltpu.CompilerParams(
                           use_tc_tiling_on_sc=False))
    def segsum(feat_hbm, edges_hbm, *outs_and_scratch):
        if with_cnt:
            (sum_out, cnt_out, acc_sp, cnt_sp) = outs_and_scratch
        else:
            (sum_out, acc_sp) = outs_and_scratch
        c = lax.axis_index("c")
        s = lax.axis_index("s")
        tile = s * NC + c
        r0 = s * RPT

        def inner(idx_v, rows_v, ones_v, pcnt_v, gsems, ssems, isems, csem):
            zero16 = jnp.zeros((16,), jnp.float32)

            def zrow(i, _):
                r = i // (D // 16)
                col = (i % (D // 16)) * 16
                rows_v[0, r, pl.ds(col, 16)] = zero16
                return _

            lax.fori_loop(0, CHUNK * (D // 16), zrow, 0)
            for k in range(NHOP):
                pltpu.sync_copy(rows_v.at[0, :CPO],
                                acc_sp.at[pl.ds(r0 + k * CPO, CPO)])

            if with_cnt:
                def zc(i, _):
                    ones_v[i] = zero16
                    return _

                lax.fori_loop(0, CHUNK, zc, 0)
                for k in range(NHOP):
                    pltpu.sync_copy(ones_v.at[:CPO],
                                    cnt_sp.at[pl.ds(r0 + k * CPO, CPO)])

                def fill16(i, _):
                    ones_v[i] = jnp.ones((16,), jnp.float32)
                    return _

                lax.fori_loop(0, CHUNK, fill16, 0)

            plsc.subcore_barrier()

            def issue_idx(ch, sl):
                pltpu.async_copy(edges_hbm.at[tile, ch], idx_v.at[sl],
                                 isems.at[sl])

            def issue_gather(ch, sl, isl):
                pltpu.make_async_copy(edges_hbm.at[tile, ch],
                                      idx_v.at[isl], isems.at[isl]).wait()
                pltpu.async_copy(feat_hbm.at[idx_v.at[isl, 0]],
                                 rows_v.at[sl], gsems.at[sl])

            for k in range(LI):
                issue_idx(k, k)
            for b in range(LEAD):
                issue_gather(b, b, b)

            def outer(g0, _):
                for b in range(NI):
                    ch = g0 * NI + b
                    sb = b % NBUF
                    s2 = (b + LEAD) % NBUF
                    i2 = (b + LEAD) % NI
                    i4 = (b + LI) % NI

                    @pl.when(ch + LI < NCH)
                    def _issue_next_idx():
                        issue_idx(ch + LI, i4)

                    @pl.when(ch >= LEAD)
                    def _wait_prev_scatter():
                        pltpu.make_async_copy(
                            rows_v.at[s2], acc_sp.at[idx_v.at[b, 1]],
                            ssems.at[s2]).wait()

                    @pl.when(ch + LEAD < NCH)
                    def _issue_next_gather():
                        issue_gather(ch + LEAD, s2, i2)

                    pltpu.make_async_copy(feat_hbm.at[idx_v.at[b, 0]],
                                          rows_v.at[sb], gsems.at[sb]).wait()
                    if with_cnt:
                        pltpu.async_copy(ones_v, cnt_sp.at[idx_v.at[b, 1]],
                                         csem, add=True)
                    pltpu.async_copy(rows_v.at[sb], acc_sp.at[idx_v.at[b, 1]],
                                     ssems.at[sb], add=True)
                return _

            lax.fori_loop(0, NCH // NI, outer, 0)

            for i in range(LEAD):
                sl = (NCH - LEAD + i) % NBUF
                pltpu.make_async_copy(rows_v.at[sl], acc_sp.at[idx_v.at[0, 1]],
                                      ssems.at[sl]).wait()
            if with_cnt:
                def drain_cnt(i, _):
                    pltpu.make_async_copy(ones_v, cnt_sp.at[idx_v.at[0, 1]],
                                          csem).wait()
                    return _

                lax.fori_loop(0, NCH, drain_cnt, 0)

            plsc.subcore_barrier()

            for k in range(NHOP):
                sl = k % NBUF
                pltpu.sync_copy(acc_sp.at[pl.ds(r0 + k * CPO, CPO)],
                                rows_v.at[sl, :CPO])
                pltpu.sync_copy(rows_v.at[sl, :CPO],
                                sum_out.at[c, pl.ds(r0 + k * CPO, CPO)])

            if with_cnt:
                for k in range(2):
                    half = RPT // 2
                    pltpu.sync_copy(
                        cnt_sp.at[pl.ds(r0 + k * half, half), pl.ds(0, 1)],
                        pcnt_v)
                    pltpu.sync_copy(pcnt_v,
                                    cnt_out.at[c, pl.ds(r0 + k * half, half)])

        pl.run_scoped(
            inner,
            pltpu.VMEM((NI, 2, CHUNK), jnp.int32),
            pltpu.VMEM((NBUF, CHUNK, D), jnp.float32),
            pltpu.VMEM((CHUNK, 16), jnp.float32),
            pltpu.VMEM((RPT // 2, 1), jnp.float32),
            pltpu.SemaphoreType.DMA((NBUF,)),
            pltpu.SemaphoreType.DMA((NBUF,)),
            pltpu.SemaphoreType.DMA((NI,)),
            pltpu.SemaphoreType.DMA,
        )

    return segsum


_R = 2000


def _tc_root(x_ref, wr, b, xr_ref):
    xr_ref[...] = (lax.dot_general(x_ref[...], wr[...],
                                   (((1,), (1,)), ((), ())),
                                   preferred_element_type=jnp.float32)
                   + b[...])


def _tc_layer1(sa, sb, ca, cb, xr_ref, wl, h_ref):
    cnt = ca[...] + cb[...]
    agg = (sa[...] + sb[...]) / jnp.maximum(cnt, 1.0)
    z = (lax.dot_general(agg, wl[...], (((1,), (1,)), ((), ())),
                         preferred_element_type=jnp.float32)
         + xr_ref[...])
    h_ref[...] = jnp.where(z >= 0, z, 0.01 * z)


def _tc_layer2(sa, sb, ca, cb, xr_ref, wl, wo, bo, y_ref):
    cnt = ca[...] + cb[...]
    agg = (sa[...] + sb[...]) / jnp.maximum(cnt, 1.0)
    z = (lax.dot_general(agg, wl[...], (((1,), (1,)), ((), ())),
                         preferred_element_type=jnp.float32)
         + xr_ref[...])
    h2 = jnp.where(z >= 0, z, 0.01 * z)
    y_ref[...] = jnp.sum(h2 * wo[...], axis=1, keepdims=True) + bo[0, 0]


def _row_block(i):
    return (i, 0)


def _whole(i):
    return (0, 0)


_bs_rows = pl.BlockSpec((_R, D), _row_block)
_bs_cnt = pl.BlockSpec((_R, 1), _row_block)
_bs_w = pl.BlockSpec((D, D), _whole)
_bs_b = pl.BlockSpec((1, D), _whole)


def kernel(x, edge_index, W1l, b1, W1r, W2l, b2, W2r, Wout, bout):
    src = edge_index[0]
    dst = edge_index[1]
    npad = EPAD - E
    ppos = jnp.arange(npad, dtype=jnp.int32)
    src_p = jnp.concatenate([src, ppos % N])
    dst_p = jnp.concatenate([dst, N + (ppos % 16)])

    def chunked(nch, ch):
        return jnp.stack([src_p.reshape(NW, nch, ch),
                          dst_p.reshape(NW, nch, ch)], axis=2)

    edges1 = chunked(NCH1, CH1)
    edges2 = chunked(NCH2, CH2)

    def root_mm(feat, wr, b):
        return pl.pallas_call(
            _tc_root,
            grid=(N // _R,),
            in_specs=[_bs_rows, _bs_w, _bs_b],
            out_specs=_bs_rows,
            out_shape=jax.ShapeDtypeStruct((N, D), jnp.float32),
        )(feat, wr, b.reshape(1, D))

    xr1 = root_mm(x, W1r, b1)
    s1, cpk = _make_segsum(True, CH1, NCH1)(x, edges1)

    c16a = cpk[0]
    c16b = cpk[1]

    h = pl.pallas_call(
        _tc_layer1,
        grid=(N // _R,),
        in_specs=[_bs_rows, _bs_rows, _bs_cnt, _bs_cnt, _bs_rows, _bs_w],
        out_specs=_bs_rows,
        out_shape=jax.ShapeDtypeStruct((N, D), jnp.float32),
    )(s1[0], s1[1], c16a, c16b, xr1, W1l)

    xr2 = root_mm(h, W2r, b2)
    (s2,) = _make_segsum(False, CH2, NCH2)(h, edges2)

    y = pl.pallas_call(
        _tc_layer2,
        grid=(N // _R,),
        in_specs=[_bs_rows, _bs_rows, _bs_cnt, _bs_cnt, _bs_rows, _bs_w,
                  pl.BlockSpec((1, D), _whole), pl.BlockSpec((1, 1), _whole)],
        out_specs=pl.BlockSpec((_R, 1), _row_block),
        out_shape=jax.ShapeDtypeStruct((N, 1), jnp.float32),
    )(s2[0], s2[1], c16a, c16b, xr2, W2l, Wout, bout.reshape(1, 1))

    return y.reshape(N)


# speedup vs baseline: 1.0035x; 1.0035x over previous
import functools

import jax
import jax.numpy as jnp
from jax import lax
from jax.experimental import pallas as pl
from jax.experimental.pallas import tpu as pltpu
from jax.experimental.pallas import tpu_sc as plsc

N = 10000
E = 320000
D = 128

NC = 2
NS = 16
NW = NC * NS

NBUF = 4
LEAD = 2
NI = 8
LI = 4
EPT = 10240
EPAD = NW * EPT
CH1, NCH1 = 64, 160
CH2, NCH2 = 64, 160

RPT = 640
NPAD = NS * RPT
CPO = 64
NHOP = RPT // CPO


@functools.lru_cache(maxsize=None)
def _make_segsum(with_cnt: bool, CHUNK: int, NCH: int):
    _mesh = plsc.VectorSubcoreMesh(core_axis_name="c", subcore_axis_name="s",
                                   num_cores=NC, num_subcores=NS)
    out_type = [jax.ShapeDtypeStruct((NC, NPAD, D), jnp.float32)]
    scratch = [
        pltpu.VMEM_SHARED((NPAD, D), jnp.float32),
    ]
    if with_cnt:
        out_type.append(jax.ShapeDtypeStruct((NC, NPAD, 1), jnp.float32))
        scratch.append(pltpu.VMEM_SHARED((NPAD, 16), jnp.float32))

    @functools.partial(pl.kernel, out_type=out_type, mesh=_mesh,
                       scratch_types=scratch,
                       compiler_params=pltpu.CompilerParams(
                           use_tc_tiling_on_sc=False))
    def segsum(feat_hbm, edges_hbm, *outs_and_scratch):
        if with_cnt:
            (sum_out, cnt_out, acc_sp, cnt_sp) = outs_and_scratch
        else:
            (sum_out, acc_sp) = outs_and_scratch
        c = lax.axis_index("c")
        s = lax.axis_index("s")
        tile = s * NC + c
        r0 = s * RPT

        def inner(idx_v, rows_v, ones_v, pcnt_v, gsems, ssems, isems, csem):
            zero16 = jnp.zeros((16,), jnp.float32)

            def zrow(i, _):
                r = i // (D // 16)
                col = (i % (D // 16)) * 16
                rows_v[0, r, pl.ds(col, 16)] = zero16
                return _

            lax.fori_loop(0, CHUNK * (D // 16), zrow, 0)
            for k in range(NHOP):
                pltpu.sync_copy(rows_v.at[0, :CPO],
                                acc_sp.at[pl.ds(r0 + k * CPO, CPO)])

            if with_cnt:
                def zc(i, _):
                    ones_v[i] = zero16
                    return _

                lax.fori_loop(0, CHUNK, zc, 0)
                for k in range(NHOP):
                    pltpu.sync_copy(ones_v.at[:CPO],
                                    cnt_sp.at[pl.ds(r0 + k * CPO, CPO)])

                def fill16(i, _):
                    ones_v[i] = jnp.ones((16,), jnp.float32)
                    return _

                lax.fori_loop(0, CHUNK, fill16, 0)

            plsc.subcore_barrier()

            def issue_idx(ch, sl):
                pltpu.async_copy(edges_hbm.at[tile, ch], idx_v.at[sl],
                                 isems.at[sl])

            def issue_gather(ch, sl, isl):
                pltpu.make_async_copy(edges_hbm.at[tile, ch],
                                      idx_v.at[isl], isems.at[isl]).wait()
                pltpu.async_copy(feat_hbm.at[idx_v.at[isl, 0]],
                                 rows_v.at[sl], gsems.at[sl])

            for k in range(LI):
                issue_idx(k, k)
            for b in range(LEAD):
                issue_gather(b, b, b)

            def outer(g0, _):
                for b in range(NI):
                    ch = g0 * NI + b
                    sb = b % NBUF
                    s2 = (b + LEAD) % NBUF
                    i2 = (b + LEAD) % NI
                    i4 = (b + LI) % NI

                    @pl.when(ch + LI < NCH)
                    def _issue_next_idx():
                        issue_idx(ch + LI, i4)

                    @pl.when(ch >= LEAD)
                    def _wait_prev_scatter():
                        pltpu.make_async_copy(
                            rows_v.at[s2], acc_sp.at[idx_v.at[b, 1]],
                            ssems.at[s2]).wait()

                    @pl.when(ch + LEAD < NCH)
                    def _issue_next_gather():
                        issue_gather(ch + LEAD, s2, i2)

                    pltpu.make_async_copy(feat_hbm.at[idx_v.at[b, 0]],
                                          rows_v.at[sb], gsems.at[sb]).wait()
                    if with_cnt:
                        pltpu.async_copy(ones_v, cnt_sp.at[idx_v.at[b, 1]],
                                         csem, add=True)
                    pltpu.async_copy(rows_v.at[sb], acc_sp.at[idx_v.at[b, 1]],
                                     ssems.at[sb], add=True)
                return _

            lax.fori_loop(0, NCH // NI, outer, 0)

            for i in range(LEAD):
                sl = (NCH - LEAD + i) % NBUF
                pltpu.make_async_copy(rows_v.at[sl], acc_sp.at[idx_v.at[0, 1]],
                                      ssems.at[sl]).wait()
            if with_cnt:
                def drain_cnt(i, _):
                    pltpu.make_async_copy(ones_v, cnt_sp.at[idx_v.at[0, 1]],
                                          csem).wait()
                    return _

                lax.fori_loop(0, NCH, drain_cnt, 0)

            plsc.subcore_barrier()

            for k in range(NHOP):
                sl = k % NBUF
                pltpu.sync_copy(acc_sp.at[pl.ds(r0 + k * CPO, CPO)],
                                rows_v.at[sl, :CPO])
                pltpu.sync_copy(rows_v.at[sl, :CPO],
                                sum_out.at[c, pl.ds(r0 + k * CPO, CPO)])

            if with_cnt:
                for k in range(2):
                    half = RPT // 2
                    pltpu.sync_copy(
                        cnt_sp.at[pl.ds(r0 + k * half, half), pl.ds(0, 1)],
                        pcnt_v)
                    pltpu.sync_copy(pcnt_v,
                                    cnt_out.at[c, pl.ds(r0 + k * half, half)])

        pl.run_scoped(
            inner,
            pltpu.VMEM((NI, 2, CHUNK), jnp.int32),
            pltpu.VMEM((NBUF, CHUNK, D), jnp.float32),
            pltpu.VMEM((CHUNK, 16), jnp.float32),
            pltpu.VMEM((RPT // 2, 1), jnp.float32),
            pltpu.SemaphoreType.DMA((NBUF,)),
            pltpu.SemaphoreType.DMA((NBUF,)),
            pltpu.SemaphoreType.DMA((NI,)),
            pltpu.SemaphoreType.DMA,
        )

    return segsum


_R = 2000


def _tc_root(x_ref, wr, b, xr_ref):
    xr_ref[...] = (lax.dot_general(x_ref[...], wr[...],
                                   (((1,), (1,)), ((), ())),
                                   preferred_element_type=jnp.float32)
                   + b[...])


def _tc_layer1(sa, sb, ca, cb, xr_ref, wl, h_ref):
    cnt = ca[...] + cb[...]
    agg = (sa[...] + sb[...]) / jnp.maximum(cnt, 1.0)
    z = (lax.dot_general(agg, wl[...], (((1,), (1,)), ((), ())),
                         preferred_element_type=jnp.float32)
         + xr_ref[...])
    h_ref[...] = jnp.where(z >= 0, z, 0.01 * z)


def _tc_layer2(sa, sb, ca, cb, xr_ref, wl, wo, bo, y_ref):
    cnt = ca[...] + cb[...]
    agg = (sa[...] + sb[...]) / jnp.maximum(cnt, 1.0)
    z = (lax.dot_general(agg, wl[...], (((1,), (1,)), ((), ())),
                         preferred_element_type=jnp.float32)
         + xr_ref[...])
    h2 = jnp.where(z >= 0, z, 0.01 * z)
    y_ref[...] = jnp.sum(h2 * wo[...], axis=1, keepdims=True) + bo[0, 0]


def _row_block(i):
    return (i, 0)


def _whole(i):
    return (0, 0)


_bs_rows = pl.BlockSpec((_R, D), _row_block)
_bs_cnt = pl.BlockSpec((_R, 1), _row_block)
_bs_w = pl.BlockSpec((D, D), _whole)
_bs_b = pl.BlockSpec((1, D), _whole)


def kernel(x, edge_index, W1l, b1, W1r, W2l, b2, W2r, Wout, bout):
    src = edge_index[0]
    dst = edge_index[1]
    npad = EPAD - E
    ppos = jnp.arange(npad, dtype=jnp.int32)
    src_p = jnp.concatenate([src, ppos % N])
    dst_p = jnp.concatenate([dst, N + (ppos % 16)])

    def chunked(nch, ch):
        return jnp.stack([src_p.reshape(NW, nch, ch),
                          dst_p.reshape(NW, nch, ch)], axis=2)

    edges1 = chunked(NCH1, CH1)
    edges2 = edges1 if (CH2, NCH2) == (CH1, NCH1) else chunked(NCH2, CH2)

    def root_mm(feat, wr, b):
        return pl.pallas_call(
            _tc_root,
            grid=(N // _R,),
            in_specs=[_bs_rows, _bs_w, _bs_b],
            out_specs=_bs_rows,
            out_shape=jax.ShapeDtypeStruct((N, D), jnp.float32),
        )(feat, wr, b.reshape(1, D))

    xr1 = root_mm(x, W1r, b1)
    s1, cpk = _make_segsum(True, CH1, NCH1)(x, edges1)

    c16a = cpk[0]
    c16b = cpk[1]

    h = pl.pallas_call(
        _tc_layer1,
        grid=(N // _R,),
        in_specs=[_bs_rows, _bs_rows, _bs_cnt, _bs_cnt, _bs_rows, _bs_w],
        out_specs=_bs_rows,
        out_shape=jax.ShapeDtypeStruct((N, D), jnp.float32),
    )(s1[0], s1[1], c16a, c16b, xr1, W1l)

    xr2 = root_mm(h, W2r, b2)
    (s2,) = _make_segsum(False, CH2, NCH2)(h, edges2)

    y = pl.pallas_call(
        _tc_layer2,
        grid=(N // _R,),
        in_specs=[_bs_rows, _bs_rows, _bs_cnt, _bs_cnt, _bs_rows, _bs_w,
                  pl.BlockSpec((1, D), _whole), pl.BlockSpec((1, 1), _whole)],
        out_specs=pl.BlockSpec((_R, 1), _row_block),
        out_shape=jax.ShapeDtypeStruct((N, 1), jnp.float32),
    )(s2[0], s2[1], c16a, c16b, xr2, W2l, Wout, bout.reshape(1, 1))

    return y.reshape(N)


# speedup vs baseline: 1.0058x; 1.0023x over previous
import functools

import jax
import jax.numpy as jnp
from jax import lax
from jax.experimental import pallas as pl
from jax.experimental.pallas import tpu as pltpu
from jax.experimental.pallas import tpu_sc as plsc

N = 10000
E = 320000
D = 128

NC = 2
NS = 16
NW = NC * NS

NBUF = 4
LEAD = 2
NI = 8
LI = 4
EPT = 10240
EPAD = NW * EPT
CH1, NCH1 = 64, 160
CH2, NCH2 = 64, 160

RPT = 640
NPAD = NS * RPT
CPO = 64
NHOP = RPT // CPO


@functools.lru_cache(maxsize=None)
def _make_segsum(with_cnt: bool, CHUNK: int, NCH: int):
    _mesh = plsc.VectorSubcoreMesh(core_axis_name="c", subcore_axis_name="s",
                                   num_cores=NC, num_subcores=NS)
    out_type = [jax.ShapeDtypeStruct((NC, NPAD, D), jnp.float32)]
    scratch = [
        pltpu.VMEM_SHARED((NPAD, D), jnp.float32),
    ]
    if with_cnt:
        out_type.append(jax.ShapeDtypeStruct((NC, NPAD, 1), jnp.float32))
        scratch.append(pltpu.VMEM_SHARED((NPAD, 16), jnp.float32))

    @functools.partial(pl.kernel, out_type=out_type, mesh=_mesh,
                       scratch_types=scratch,
                       compiler_params=pltpu.CompilerParams(
                           use_tc_tiling_on_sc=False))
    def segsum(feat_hbm, edges_hbm, *outs_and_scratch):
        if with_cnt:
            (sum_out, cnt_out, acc_sp, cnt_sp) = outs_and_scratch
        else:
            (sum_out, acc_sp) = outs_and_scratch
        c = lax.axis_index("c")
        s = lax.axis_index("s")
        tile = s * NC + c
        r0 = s * RPT

        def inner(idx_v, rows_v, ones_v, pcnt_v, gsems, ssems, isems, csem):
            zero16 = jnp.zeros((16,), jnp.float32)

            def zrow(i, _):
                r = i // (D // 16)
                col = (i % (D // 16)) * 16
                rows_v[0, r, pl.ds(col, 16)] = zero16
                return _

            lax.fori_loop(0, CHUNK * (D // 16), zrow, 0)
            for k in range(NHOP):
                pltpu.sync_copy(rows_v.at[0, :CPO],
                                acc_sp.at[pl.ds(r0 + k * CPO, CPO)])

            if with_cnt:
                def zc(i, _):
                    ones_v[i] = zero16
                    return _

                lax.fori_loop(0, CHUNK, zc, 0)
                for k in range(NHOP):
                    pltpu.sync_copy(ones_v.at[:CPO],
                                    cnt_sp.at[pl.ds(r0 + k * CPO, CPO)])

                def fill16(i, _):
                    ones_v[i] = jnp.ones((16,), jnp.float32)
                    return _

                lax.fori_loop(0, CHUNK, fill16, 0)

            plsc.subcore_barrier()

            def issue_idx(ch, sl):
                pltpu.async_copy(edges_hbm.at[tile, ch], idx_v.at[sl],
                                 isems.at[sl])

            def issue_gather(ch, sl, isl):
                pltpu.make_async_copy(edges_hbm.at[tile, ch],
                                      idx_v.at[isl], isems.at[isl]).wait()
                pltpu.async_copy(feat_hbm.at[idx_v.at[isl, 0]],
                                 rows_v.at[sl], gsems.at[sl])

            for k in range(LI):
                issue_idx(k, k)
            for b in range(LEAD):
                issue_gather(b, b, b)

            def outer(g0, _):
                for b in range(NI):
                    ch = g0 * NI + b
                    sb = b % NBUF
                    s2 = (b + LEAD) % NBUF
                    i2 = (b + LEAD) % NI
                    i4 = (b + LI) % NI

                    @pl.when(ch + LI < NCH)
                    def _issue_next_idx():
                        issue_idx(ch + LI, i4)

                    @pl.when(ch >= LEAD)
                    def _wait_prev_scatter():
                        pltpu.make_async_copy(
                            rows_v.at[s2], acc_sp.at[idx_v.at[b, 1]],
                            ssems.at[s2]).wait()

                    @pl.when(ch + LEAD < NCH)
                    def _issue_next_gather():
                        issue_gather(ch + LEAD, s2, i2)

                    pltpu.make_async_copy(feat_hbm.at[idx_v.at[b, 0]],
                                          rows_v.at[sb], gsems.at[sb]).wait()
                    if with_cnt:
                        pltpu.async_copy(ones_v, cnt_sp.at[idx_v.at[b, 1]],
                                         csem, add=True)
                    pltpu.async_copy(rows_v.at[sb], acc_sp.at[idx_v.at[b, 1]],
                                     ssems.at[sb], add=True)
                return _

            lax.fori_loop(0, NCH // NI, outer, 0)

            for i in range(LEAD):
                sl = (NCH - LEAD + i) % NBUF
                pltpu.make_async_copy(rows_v.at[sl], acc_sp.at[idx_v.at[0, 1]],
                                      ssems.at[sl]).wait()
            if with_cnt:
                def drain_cnt(i, _):
                    pltpu.make_async_copy(ones_v, cnt_sp.at[idx_v.at[0, 1]],
                                          csem).wait()
                    return _

                lax.fori_loop(0, NCH, drain_cnt, 0)

            plsc.subcore_barrier()

            for k in range(NHOP):
                sl = k % NBUF
                pltpu.sync_copy(acc_sp.at[pl.ds(r0 + k * CPO, CPO)],
                                rows_v.at[sl, :CPO])
                pltpu.sync_copy(rows_v.at[sl, :CPO],
                                sum_out.at[c, pl.ds(r0 + k * CPO, CPO)])

            if with_cnt:
                for k in range(2):
                    half = RPT // 2
                    pltpu.sync_copy(
                        cnt_sp.at[pl.ds(r0 + k * half, half), pl.ds(0, 1)],
                        pcnt_v)
                    pltpu.sync_copy(pcnt_v,
                                    cnt_out.at[c, pl.ds(r0 + k * half, half)])

        pl.run_scoped(
            inner,
            pltpu.VMEM((NI, 2, CHUNK), jnp.int32),
            pltpu.VMEM((NBUF, CHUNK, D), jnp.float32),
            pltpu.VMEM((CHUNK, 16), jnp.float32),
            pltpu.VMEM((RPT // 2, 1), jnp.float32),
            pltpu.SemaphoreType.DMA((NBUF,)),
            pltpu.SemaphoreType.DMA((NBUF,)),
            pltpu.SemaphoreType.DMA((NI,)),
            pltpu.SemaphoreType.DMA,
        )

    return segsum


_R = 2000


def _tc_root(x_ref, wr, b, xr_ref):
    xr_ref[...] = (lax.dot_general(x_ref[...], wr[...],
                                   (((1,), (1,)), ((), ())),
                                   preferred_element_type=jnp.float32)
                   + b[...])


def _tc_layer1(sa, sb, ca, cb, xr_ref, wl, h_ref):
    cnt = ca[...] + cb[...]
    agg = (sa[...] + sb[...]) / jnp.maximum(cnt, 1.0)
    z = (lax.dot_general(agg, wl[...], (((1,), (1,)), ((), ())),
                         preferred_element_type=jnp.float32)
         + xr_ref[...])
    h_ref[...] = jnp.where(z >= 0, z, 0.01 * z)


def _tc_layer2(sa, sb, ca, cb, xr_ref, wl, wo, bo, y_ref):
    cnt = ca[...] + cb[...]
    agg = (sa[...] + sb[...]) / jnp.maximum(cnt, 1.0)
    z = (lax.dot_general(agg, wl[...], (((1,), (1,)), ((), ())),
                         preferred_element_type=jnp.float32)
         + xr_ref[...])
    h2 = jnp.where(z >= 0, z, 0.01 * z)
    y_ref[...] = jnp.sum(h2 * wo[...], axis=1, keepdims=True) + bo[0, 0]


def _row_block(i):
    return (i, 0)


def _whole(i):
    return (0, 0)


_bs_rows = pl.BlockSpec((_R, D), _row_block)
_bs_cnt = pl.BlockSpec((_R, 1), _row_block)
_bs_w = pl.BlockSpec((D, D), _whole)
_bs_b = pl.BlockSpec((1, D), _whole)


def kernel(x, edge_index, W1l, b1, W1r, W2l, b2, W2r, Wout, bout):
    src = edge_index[0]
    dst = edge_index[1]
    npad = EPAD - E
    ppos = jnp.arange(npad, dtype=jnp.int32)
    src_p = jnp.concatenate([src, ppos % N])
    dst_p = jnp.concatenate([dst, N + (ppos % 16)])

    def chunked(nch, ch):
        return jnp.stack([src_p.reshape(NW, nch, ch),
                          dst_p.reshape(NW, nch, ch)], axis=2)

    edges1 = chunked(NCH1, CH1)
    edges2 = edges1

    def root_mm(feat, wr, b):
        return pl.pallas_call(
            _tc_root,
            grid=(N // _R,),
            in_specs=[_bs_rows, _bs_w, _bs_b],
            out_specs=_bs_rows,
            out_shape=jax.ShapeDtypeStruct((N, D), jnp.float32),
        )(feat, wr, b.reshape(1, D))

    xr1 = root_mm(x, W1r, b1)
    s1, cpk = _make_segsum(True, CH1, NCH1)(x, edges1)

    c16a = cpk[0]
    c16b = cpk[1]

    h = pl.pallas_call(
        _tc_layer1,
        grid=(N // _R,),
        in_specs=[_bs_rows, _bs_rows, _bs_cnt, _bs_cnt, _bs_rows, _bs_w],
        out_specs=_bs_rows,
        out_shape=jax.ShapeDtypeStruct((N, D), jnp.float32),
    )(s1[0], s1[1], c16a, c16b, xr1, W1l)

    xr2 = root_mm(h, W2r, b2)
    (s2,) = _make_segsum(False, CH2, NCH2)(h, edges2)

    y = pl.pallas_call(
        _tc_layer2,
        grid=(N // _R,),
        in_specs=[_bs_rows, _bs_rows, _bs_cnt, _bs_cnt, _bs_rows, _bs_w,
                  pl.BlockSpec((1, D), _whole), pl.BlockSpec((1, 1), _whole)],
        out_specs=pl.BlockSpec((_R, 1), _row_block),
        out_shape=jax.ShapeDtypeStruct((N, 1), jnp.float32),
    )(s2[0], s2[1], c16a, c16b, xr2, W2l, Wout, bout.reshape(1, 1))

    return y.reshape(N)
